# baseline (speedup 1.0000x reference)
import functools

import jax
import jax.numpy as jnp
from jax import lax
from jax.experimental import pallas as pl
from jax.experimental.pallas import tpu as pltpu
from jax.experimental.pallas import tpu_sc as plsc

D_MODEL = 64
SCALE = 8.0
LANES = 16

NUM_CORES = 2
NUM_SUBCORES = 16
NUM_WORKERS = NUM_CORES * NUM_SUBCORES

NB_TOTAL = 4096
SEQ = 50
NB_PER_WORKER = NB_TOTAL // NUM_WORKERS
NB_CHUNK = 8
N_CHUNKS = NB_PER_WORKER // NB_CHUNK


def _make_gather_kernel():
    mesh = plsc.VectorSubcoreMesh(core_axis_name="c", subcore_axis_name="s")

    @functools.partial(
        pl.kernel,
        mesh=mesh,
        compiler_params=pltpu.CompilerParams(use_tc_tiling_on_sc=False),
        out_type=jax.ShapeDtypeStruct((NB_TOTAL, SEQ, D_MODEL), jnp.float32),
        scratch_types=[
            pltpu.VMEM((NB_PER_WORKER, 4 * LANES), jnp.int32),
            pltpu.VMEM((NB_CHUNK, SEQ, D_MODEL), jnp.float32),
            pltpu.VMEM((NB_CHUNK, SEQ, D_MODEL), jnp.float32),
            pltpu.SemaphoreType.DMA,
            pltpu.SemaphoreType.DMA,
            pltpu.SemaphoreType.DMA,
            pltpu.SemaphoreType.DMA,
        ],
    )
    def gather_scale(
        table_hbm, x_hbm, out_hbm, idx_v, rows_a, rows_b, gsem_a, gsem_b, wsem_a, wsem_b
    ):
        wid = lax.axis_index("s") * NUM_CORES + lax.axis_index("c")
        b0 = wid * NB_PER_WORKER
        rows = (rows_a, rows_b)
        gsem = (gsem_a, gsem_b)
        wsem = (wsem_a, wsem_b)

        pltpu.sync_copy(x_hbm.at[pl.ds(b0, NB_PER_WORKER), :], idx_v)

        def fire_gather(c, s):
            @pl.loop(0, NB_CHUNK)
            def _(q):
                qq = c * NB_CHUNK + q
                for g in range(3):
                    v = idx_v[qq, pl.ds(g * LANES, LANES)]
                    pltpu.async_copy(
                        table_hbm.at[v],
                        rows[s].at[q, pl.ds(g * LANES, LANES), :],
                        gsem[s],
                    )
                v = idx_v[qq, pl.ds(3 * LANES, LANES)]
                for t in range(SEQ - 3 * LANES):
                    pltpu.async_copy(
                        table_hbm.at[pl.ds(v[t], 1), :],
                        rows[s].at[q, pl.ds(3 * LANES + t, 1), :],
                        gsem[s],
                    )

        def drain_gather(s):
            @pl.loop(0, NB_CHUNK)
            def _(q):
                for g in range(3):
                    pltpu.make_async_copy(
                        table_hbm.at[pl.ds(0, LANES), :],
                        rows[s].at[0, pl.ds(g * LANES, LANES), :],
                        gsem[s],
                    ).wait()
                for t in range(SEQ - 3 * LANES):
                    pltpu.make_async_copy(
                        table_hbm.at[pl.ds(0, 1), :],
                        rows[s].at[0, pl.ds(3 * LANES + t, 1), :],
                        gsem[s],
                    ).wait()

        def scale(s):
            @pl.loop(0, NB_CHUNK)
            def _(q):
                @pl.loop(0, SEQ)
                def _(r):
                    for l in range(D_MODEL // LANES):
                        slc = (q, pl.ds(r, 1), pl.ds(l * LANES, LANES))
                        rows[s].at[*slc][...] = rows[s].at[*slc][...] * SCALE

        def fire_wb(c, s):
            pltpu.async_copy(rows[s], out_hbm.at[pl.ds(b0 + c * NB_CHUNK, NB_CHUNK)], wsem[s])

        def drain_wb(c, s):
            pltpu.make_async_copy(
                rows[s], out_hbm.at[pl.ds(b0 + c * NB_CHUNK, NB_CHUNK)], wsem[s]
            ).wait()

        for c in range(N_CHUNKS):
            s = c & 1
            if c >= 2:
                drain_wb(c - 2, s)
            fire_gather(c, s)
            if c >= 1:
                o = 1 - s
                drain_gather(o)
                scale(o)
                fire_wb(c - 1, o)
        s_last = (N_CHUNKS - 1) & 1
        drain_gather(s_last)
        scale(s_last)
        fire_wb(N_CHUNKS - 1, s_last)
        drain_wb(N_CHUNKS - 2, 1 - s_last)
        drain_wb(N_CHUNKS - 1, s_last)

    return gather_scale


_gather_scale = _make_gather_kernel()


@jax.jit
def kernel(x, table):
    xp = jnp.pad(x.astype(jnp.int32), ((0, 0), (0, 4 * LANES - SEQ)))
    return _gather_scale(table, xp)


# speedup vs baseline: 1.3613x; 1.3613x over previous
import functools

import jax
import jax.numpy as jnp
from jax import lax
from jax.experimental import pallas as pl
from jax.experimental.pallas import tpu as pltpu
from jax.experimental.pallas import tpu_sc as plsc

D_MODEL = 64
SCALE = 8.0
LANES = 16

NUM_CORES = 2
NUM_SUBCORES = 16
NUM_WORKERS = NUM_CORES * NUM_SUBCORES

NB_TOTAL = 4096
SEQ = 50
NB_PER_WORKER = NB_TOTAL // NUM_WORKERS
NB_CHUNK = 8
N_CHUNKS = NB_PER_WORKER // NB_CHUNK


def _make_gather_kernel():
    mesh = plsc.VectorSubcoreMesh(core_axis_name="c", subcore_axis_name="s")

    @functools.partial(
        pl.kernel,
        mesh=mesh,
        out_type=jax.ShapeDtypeStruct((NB_TOTAL, SEQ, D_MODEL), jnp.float32),
        scratch_types=[
            pltpu.VMEM((NB_PER_WORKER, 4 * LANES), jnp.int32),
            pltpu.VMEM((NB_CHUNK, SEQ, D_MODEL), jnp.float32),
            pltpu.VMEM((NB_CHUNK, SEQ, D_MODEL), jnp.float32),
            pltpu.SemaphoreType.DMA,
            pltpu.SemaphoreType.DMA,
            pltpu.SemaphoreType.DMA,
            pltpu.SemaphoreType.DMA,
        ],
    )
    def gather_scale(
        table_hbm, x_hbm, out_hbm, idx_v, rows_a, rows_b, gsem_a, gsem_b, wsem_a, wsem_b
    ):
        wid = lax.axis_index("s") * NUM_CORES + lax.axis_index("c")
        b0 = wid * NB_PER_WORKER
        rows = (rows_a, rows_b)
        gsem = (gsem_a, gsem_b)
        wsem = (wsem_a, wsem_b)

        pltpu.sync_copy(x_hbm.at[pl.ds(b0, NB_PER_WORKER), :], idx_v)

        def fire_gather(c, s):
            @pl.loop(0, NB_CHUNK)
            def _(q):
                qq = c * NB_CHUNK + q
                for g in range(4):
                    v = idx_v[qq, pl.ds(g * LANES, LANES)]
                    for t in range(min(LANES, SEQ - g * LANES)):
                        pltpu.async_copy(
                            table_hbm.at[pl.ds(v[t], 1), :],
                            rows[s].at[q, pl.ds(g * LANES + t, 1), :],
                            gsem[s],
                        )

        def drain_gather(s):
            @pl.loop(0, NB_CHUNK * SEQ)
            def _(r):
                pltpu.make_async_copy(
                    table_hbm.at[pl.ds(0, 1), :],
                    rows[s].at[0, pl.ds(0, 1), :],
                    gsem[s],
                ).wait()

        def scale(s):
            @pl.loop(0, NB_CHUNK)
            def _(q):
                @pl.loop(0, SEQ)
                def _(r):
                    for l in range(D_MODEL // LANES):
                        slc = (q, pl.ds(r, 1), pl.ds(l * LANES, LANES))
                        rows[s].at[*slc][...] = rows[s].at[*slc][...] * SCALE

        def fire_wb(c, s):
            pltpu.async_copy(rows[s], out_hbm.at[pl.ds(b0 + c * NB_CHUNK, NB_CHUNK)], wsem[s])

        def drain_wb(c, s):
            pltpu.make_async_copy(
                rows[s], out_hbm.at[pl.ds(b0 + c * NB_CHUNK, NB_CHUNK)], wsem[s]
            ).wait()

        for c in range(N_CHUNKS):
            s = c & 1
            if c >= 2:
                drain_wb(c - 2, s)
            fire_gather(c, s)
            if c >= 1:
                o = 1 - s
                drain_gather(o)
                scale(o)
                fire_wb(c - 1, o)
        s_last = (N_CHUNKS - 1) & 1
        drain_gather(s_last)
        scale(s_last)
        fire_wb(N_CHUNKS - 1, s_last)
        drain_wb(N_CHUNKS - 2, 1 - s_last)
        drain_wb(N_CHUNKS - 1, s_last)

    return gather_scale


_gather_scale = _make_gather_kernel()


@jax.jit
def kernel(x, table):
    xp = jnp.pad(x.astype(jnp.int32), ((0, 0), (0, 4 * LANES - SEQ)))
    return _gather_scale(table, xp)


# speedup vs baseline: 1.4509x; 1.0658x over previous
import functools

import jax
import jax.numpy as jnp
from jax import lax
from jax.experimental import pallas as pl
from jax.experimental.pallas import tpu as pltpu
from jax.experimental.pallas import tpu_sc as plsc

D_MODEL = 64
SCALE = 8.0
LANES = 16

NUM_CORES = 2
NUM_SUBCORES = 16
NUM_WORKERS = NUM_CORES * NUM_SUBCORES

B_TOTAL = 4096 * 50
B_PER_WORKER = B_TOTAL // NUM_WORKERS
IDX_ROWS_PER_WORKER = B_PER_WORKER // 128
CHUNK = 256
N_CHUNKS = B_PER_WORKER // CHUNK


def _make_gather_kernel():
    mesh = plsc.VectorSubcoreMesh(core_axis_name="c", subcore_axis_name="s")

    @functools.partial(
        pl.kernel,
        mesh=mesh,
        out_type=jax.ShapeDtypeStruct((B_TOTAL, D_MODEL), jnp.float32),
        scratch_types=[
            pltpu.VMEM((B_PER_WORKER,), jnp.int32),
            pltpu.VMEM((CHUNK, D_MODEL), jnp.float32),
            pltpu.VMEM((CHUNK, D_MODEL), jnp.float32),
            pltpu.SemaphoreType.DMA,
            pltpu.SemaphoreType.DMA,
            pltpu.SemaphoreType.DMA,
            pltpu.SemaphoreType.DMA,
        ],
    )
    def gather_scale(
        table_hbm, idx_hbm, out_hbm, idx_v, rows_a, rows_b, gsem_a, gsem_b, wsem_a, wsem_b
    ):
        wid = lax.axis_index("s") * NUM_CORES + lax.axis_index("c")
        base = wid * B_PER_WORKER
        rows = (rows_a, rows_b)
        gsem = (gsem_a, gsem_b)
        wsem = (wsem_a, wsem_b)

        pltpu.sync_copy(idx_hbm.at[pl.ds(base, B_PER_WORKER)], idx_v)

        def fire_gather(c, s):
            @pl.loop(0, CHUNK, step=LANES)
            def _(r):
                v = idx_v[pl.ds(c * CHUNK + r, LANES)]
                for t in range(LANES):
                    pltpu.async_copy(
                        table_hbm.at[pl.ds(v[t], 1), :],
                        rows[s].at[pl.ds(r + t, 1), :],
                        gsem[s],
                    )

        def drain_gather(s):
            @pl.loop(0, CHUNK)
            def _(r):
                pltpu.make_async_copy(
                    table_hbm.at[pl.ds(0, 1), :],
                    rows[s].at[pl.ds(0, 1), :],
                    gsem[s],
                ).wait()

        def scale(s):
            @pl.loop(0, CHUNK)
            def _(r):
                for l in range(D_MODEL // LANES):
                    slc = (pl.ds(r, 1), pl.ds(l * LANES, LANES))
                    rows[s].at[*slc][...] = rows[s].at[*slc][...] * SCALE

        def wb_slice(c):
            return out_hbm.at[pl.ds(pl.multiple_of(base + c * CHUNK, 8), CHUNK), :]

        def fire_wb(c, s):
            pltpu.async_copy(rows[s], wb_slice(c), wsem[s])

        def drain_wb(c, s):
            pltpu.make_async_copy(rows[s], wb_slice(c), wsem[s]).wait()

        for c in range(N_CHUNKS):
            s = c & 1
            if c >= 2:
                drain_wb(c - 2, s)
            fire_gather(c, s)
            if c >= 1:
                o = 1 - s
                drain_gather(o)
                scale(o)
                fire_wb(c - 1, o)
        s_last = (N_CHUNKS - 1) & 1
        drain_gather(s_last)
        scale(s_last)
        fire_wb(N_CHUNKS - 1, s_last)
        drain_wb(N_CHUNKS - 2, 1 - s_last)
        drain_wb(N_CHUNKS - 1, s_last)

    return gather_scale


_gather_scale = _make_gather_kernel()


@jax.jit
def kernel(x, table):
    out = _gather_scale(table, x.reshape(-1).astype(jnp.int32))
    return out.reshape(x.shape + (D_MODEL,))
